# Initial kernel scaffold; baseline (speedup 1.0000x reference)
import jax
import jax.numpy as jnp
from jax.experimental import pallas as pl


def kernel(x):
    raise NotImplementedError("write your pallas kernel here")


# speedup vs baseline: 9.2572x; 9.2572x over previous
import numpy as np
import jax
import jax.numpy as jnp
from jax.experimental import pallas as pl
from jax.experimental.pallas import tpu as pltpu

N = 8192
N1 = 64
N2 = 128
NSEQ = 64 * 32
TOPK = 5
B = 16

_HI = jax.lax.Precision.HIGHEST


def _make_tables():
    k1 = np.arange(N1)
    n1 = np.arange(N1)
    C1 = np.cos(2 * np.pi * np.outer(k1, n1) / N1).astype(np.float32)
    S1 = np.sin(2 * np.pi * np.outer(k1, n1) / N1).astype(np.float32)
    n2 = np.arange(N2)
    Ct = np.cos(2 * np.pi * np.outer(k1, n2) / N).astype(np.float32)
    St = np.sin(2 * np.pi * np.outer(k1, n2) / N).astype(np.float32)
    k2 = np.arange(N2)
    C2 = np.cos(2 * np.pi * np.outer(n2, k2) / N2).astype(np.float32)
    S2 = np.sin(2 * np.pi * np.outer(n2, k2) / N2).astype(np.float32)
    fgrid = (k1[:, None] + N1 * k2[None, :]).astype(np.float32)
    valid = ((fgrid >= 1) & (fgrid <= N // 2)).astype(np.float32)
    return C1, S1, Ct, St, C2, S2, fgrid, valid


_TABLES = _make_tables()


def _dft_decomp_kernel(x_ref, c1_ref, s1_ref, ct_ref, st_ref, c2_ref,
                       s2_ref, fg_ref, valid_ref, season_ref, trend_ref):
    X = x_ref[...]
    C1 = c1_ref[...]
    S1 = s1_ref[...]
    Ct = ct_ref[...]
    St = st_ref[...]
    C2 = c2_ref[...]
    S2 = s2_ref[...]
    fg = fg_ref[...]
    valid = valid_ref[...]

    yre_l = []
    yim_l = []
    for b in range(B):
        xb = X[b]
        yre_l.append(jax.lax.dot(C1, xb, precision=_HI)[None])
        yim_l.append(-jax.lax.dot(S1, xb, precision=_HI)[None])
    Yre = jnp.concatenate(yre_l, axis=0)
    Yim = jnp.concatenate(yim_l, axis=0)

    Ypre = Yre * Ct[None] + Yim * St[None]
    Ypim = Yim * Ct[None] - Yre * St[None]

    Ypre2 = Ypre.reshape(B * N1, N2)
    Ypim2 = Ypim.reshape(B * N1, N2)
    Zre2 = (jax.lax.dot(Ypre2, C2, precision=_HI)
            + jax.lax.dot(Ypim2, S2, precision=_HI))
    Zim2 = (jax.lax.dot(Ypim2, C2, precision=_HI)
            - jax.lax.dot(Ypre2, S2, precision=_HI))
    Zre = Zre2.reshape(B, N1, N2)
    Zim = Zim2.reshape(B, N1, N2)

    mag = jnp.where(valid[None] > 0, Zre * Zre + Zim * Zim, -1.0)

    season = jnp.zeros((B, N1, N2), jnp.float32)
    for _ in range(TOPK):
        m = jnp.max(jnp.max(mag, axis=2, keepdims=True), axis=1,
                    keepdims=True)
        sel = (mag == m).astype(jnp.float32)
        a = jnp.sum(jnp.sum(sel * Zre, axis=2, keepdims=True), axis=1,
                    keepdims=True)
        bb = jnp.sum(jnp.sum(sel * Zim, axis=2, keepdims=True), axis=1,
                     keepdims=True)
        fsel = jnp.sum(jnp.sum(sel * fg[None], axis=2, keepdims=True),
                       axis=1, keepdims=True)
        k2f = jnp.floor(fsel * (1.0 / N1))
        k1f = fsel - N1 * k2f
        eps = jnp.where(fsel == float(N // 2), 1.0, 2.0)

        k1i = k1f.reshape(B, 1).astype(jnp.int32)
        k2i = k2f.reshape(B, 1).astype(jnp.int32)
        roh = (jax.lax.broadcasted_iota(jnp.int32, (B, N1), 1)
               == k1i).astype(jnp.float32)
        coh = (jax.lax.broadcasted_iota(jnp.int32, (B, N2), 1)
               == k2i).astype(jnp.float32)
        ure = jax.lax.dot(roh, C1, precision=_HI)
        uim = jax.lax.dot(roh, S1, precision=_HI)
        twc = jax.lax.dot(roh, Ct, precision=_HI)
        tws = jax.lax.dot(roh, St, precision=_HI)
        c2v = jax.lax.dot(coh, C2, precision=_HI)
        s2v = jax.lax.dot(coh, S2, precision=_HI)
        wre = twc * c2v - tws * s2v
        wim = twc * s2v + tws * c2v
        scale = (eps * (1.0 / N)).reshape(B, 1)
        a2 = a.reshape(B, 1)
        b2 = bb.reshape(B, 1)
        cure = scale * (a2 * ure - b2 * uim)
        cuim = scale * (a2 * uim + b2 * ure)
        season = (season + cure[:, :, None] * wre[:, None, :]
                  - cuim[:, :, None] * wim[:, None, :])
        mag = jnp.where(sel > 0, -1.0, mag)

    season_ref[...] = season
    trend_ref[...] = X - season


def _run(x3, interpret=False):
    nseq = x3.shape[0]
    grid = (nseq // B,)
    tabs = [jnp.asarray(t) for t in _TABLES]
    tab_specs = [pl.BlockSpec(t.shape, lambda i: (0,) * t.ndim)
                 for t in tabs]
    season3, trend3 = pl.pallas_call(
        _dft_decomp_kernel,
        grid=grid,
        in_specs=[pl.BlockSpec((B, N1, N2), lambda i: (i, 0, 0))] + tab_specs,
        out_specs=[pl.BlockSpec((B, N1, N2), lambda i: (i, 0, 0)),
                   pl.BlockSpec((B, N1, N2), lambda i: (i, 0, 0))],
        out_shape=[jax.ShapeDtypeStruct((nseq, N1, N2), jnp.float32),
                   jax.ShapeDtypeStruct((nseq, N1, N2), jnp.float32)],
        interpret=interpret,
    )(x3, *tabs)
    return season3, trend3


def kernel(x):
    bsz, ch, n = x.shape
    x3 = x.reshape(bsz * ch, N1, N2)
    season3, trend3 = _run(x3)
    return (season3.reshape(bsz, ch, n), trend3.reshape(bsz, ch, n))
